# baseline (speedup 1.0000x reference)
import functools

import jax
import jax.numpy as jnp
from jax import lax
from jax.experimental import pallas as pl
from jax.experimental.pallas import tpu as pltpu
from jax.experimental.pallas import tpu_sc as plsc

N_NODES = 10000
N_EDGES = 320000
D = 128
N_GRAPHS = 64

CHUNK = 128
N_CORES = 2
N_SUB = 16
TILES = N_CORES * N_SUB
PER_TILE = 80
NCHUNKS = TILES * PER_TILE
E_PAD = NCHUNKS * CHUNK
ROWS_PER_SUB = 632
AGG_ROWS = N_SUB * ROWS_PER_SUB

ROW_BLK = 1000
GRID = N_NODES // ROW_BLK


IBLK = 8
PT = 160
N_BLOCKS = PT // IBLK


def _sc_agg_body(x_hbm, src_hbm, dst_hbm, z_hbm, out_hbm,
                 src_v, dst_v, rows_a, rows_b, agg_sh,
                 sem_a, sem_b, sem_i, sem_j):
    c = lax.axis_index("c")
    s = lax.axis_index("s")
    tile_base = s * PT

    @pl.when(c == 0)
    def _work():
        pltpu.sync_copy(z_hbm,
                        agg_sh.at[pl.ds(s * ROWS_PER_SUB, ROWS_PER_SUB)])

        def _pref(blk, half, sem_s, sem_d):
            off = tile_base + blk * IBLK
            pltpu.async_copy(src_hbm.at[pl.ds(off, IBLK)],
                             src_v.at[pl.ds(half, IBLK)], sem_s)
            pltpu.async_copy(dst_hbm.at[pl.ds(off, IBLK)],
                             dst_v.at[pl.ds(half, IBLK)], sem_d)

        def _wait_idx(half, sem, buf):
            pltpu.make_async_copy(src_hbm.at[pl.ds(tile_base, IBLK)],
                                  buf.at[pl.ds(half, IBLK)], sem).wait()

        _pref(0, 0, sem_i, sem_j)
        _wait_idx(0, sem_i, src_v)
        pltpu.async_copy(x_hbm.at[src_v.at[0]], rows_a, sem_a)

        def _halfblock(rb, nxt_row):
            for t in range(IBLK // 2):
                r0 = rb + 2 * t
                pltpu.make_async_copy(x_hbm.at[src_v.at[r0]], rows_a,
                                      sem_a).wait()
                pltpu.async_copy(x_hbm.at[src_v.at[r0 + 1]], rows_b, sem_b)
                pltpu.sync_copy(rows_a, agg_sh.at[dst_v.at[r0]], add=True)
                pltpu.make_async_copy(x_hbm.at[src_v.at[r0 + 1]], rows_b,
                                      sem_b).wait()
                if t == IBLK // 2 - 1:
                    _wait_idx(nxt_row, sem_i, src_v)
                    nr = nxt_row
                else:
                    nr = r0 + 2
                pltpu.async_copy(x_hbm.at[src_v.at[nr]], rows_a, sem_a)
                pltpu.sync_copy(rows_b, agg_sh.at[dst_v.at[r0 + 1]],
                                add=True)

        def body(q, carry):
            _wait_idx(0, sem_j, dst_v)
            _pref(2 * q + 1, IBLK, sem_i, sem_j)
            _halfblock(0, IBLK)
            _wait_idx(IBLK, sem_j, dst_v)
            _pref(lax.rem(2 * q + 2, N_BLOCKS), 0, sem_i, sem_j)
            _halfblock(IBLK, 0)
            return carry

        lax.fori_loop(0, PT // (2 * IBLK), body, 0)
        pltpu.make_async_copy(x_hbm.at[src_v.at[0]], rows_a, sem_a).wait()
        _wait_idx(0, sem_j, dst_v)
        plsc.subcore_barrier()

        pltpu.sync_copy(agg_sh.at[pl.ds(s * ROWS_PER_SUB, ROWS_PER_SUB)],
                        out_hbm.at[pl.ds(s * ROWS_PER_SUB, ROWS_PER_SUB)])


def _sc_agg(x, src2d, dst2d, zrows):
    f = functools.partial(
        pl.kernel,
        out_type=jax.ShapeDtypeStruct((AGG_ROWS, D), jnp.float32),
        mesh=plsc.VectorSubcoreMesh(core_axis_name="c", subcore_axis_name="s"),
        scratch_types=[
            pltpu.VMEM((2 * IBLK, CHUNK), jnp.int32),
            pltpu.VMEM((2 * IBLK, CHUNK), jnp.int32),
            pltpu.VMEM((CHUNK, D), jnp.float32),
            pltpu.VMEM((CHUNK, D), jnp.float32),
            pltpu.VMEM_SHARED((AGG_ROWS, D), jnp.float32),
            pltpu.SemaphoreType.DMA,
            pltpu.SemaphoreType.DMA,
            pltpu.SemaphoreType.DMA,
            pltpu.SemaphoreType.DMA,
        ],
    )(_sc_agg_body)
    return f(x, src2d, dst2d, zrows)


def _mlp_body(x_ref, a_ref, w1_ref, b1_ref, w2_ref, b2_ref, o_ref):
    h = x_ref[...] + a_ref[...]
    h = jnp.dot(h, w1_ref[...], preferred_element_type=jnp.float32) + b1_ref[...]
    h = jnp.maximum(h, 0.0)
    h = jnp.dot(h, w2_ref[...], preferred_element_type=jnp.float32) + b2_ref[...]
    o_ref[...] = jnp.maximum(h, 0.0)


def _mlp(x, agg, w1t, b1, w2t, b2):
    wspec = pl.BlockSpec((D, D), lambda i: (0, 0))
    bspec = pl.BlockSpec((1, D), lambda i: (0, 0))
    return pl.pallas_call(
        _mlp_body,
        grid=(GRID,),
        in_specs=[
            pl.BlockSpec((ROW_BLK, D), lambda i: (i, 0)),
            pl.BlockSpec((ROW_BLK, D), lambda i: (i, 0)),
            wspec, bspec, wspec, bspec,
        ],
        out_specs=pl.BlockSpec((ROW_BLK, D), lambda i: (i, 0)),
        out_shape=jax.ShapeDtypeStruct((N_NODES, D), jnp.float32),
    )(x, agg, w1t, b1, w2t, b2)


def _pool_body(x_ref, a_ref, seg_ref, w3_ref, b3_ref, w4_ref, b4_ref,
               wf_ref, bf_ref, o_ref, acc_ref):
    i = pl.program_id(0)
    h = x_ref[...] + a_ref[...]
    h = jnp.dot(h, w3_ref[...], preferred_element_type=jnp.float32) + b3_ref[...]
    h = jnp.maximum(h, 0.0)
    h = jnp.dot(h, w4_ref[...], preferred_element_type=jnp.float32) + b4_ref[...]
    h = jnp.maximum(h, 0.0)
    seg = seg_ref[0, 0, :]
    onehot = (seg[:, None] == lax.broadcasted_iota(jnp.int32, (1, N_GRAPHS), 1))
    onehot = onehot.astype(jnp.float32)
    p = lax.dot_general(onehot, h, (((0,), (0,)), ((), ())),
                        preferred_element_type=jnp.float32)

    @pl.when(i == 0)
    def _():
        acc_ref[...] = p

    @pl.when(i > 0)
    def _():
        acc_ref[...] = acc_ref[...] + p

    @pl.when(i == GRID - 1)
    def _():
        out = jnp.dot(acc_ref[...], wf_ref[...],
                      preferred_element_type=jnp.float32) + bf_ref[...]
        o_ref[...] = jnp.maximum(out, 0.0)


def _pool(h1, agg, seg3d, w3t, b3, w4t, b4, wft, bf):
    wspec = pl.BlockSpec((D, D), lambda i: (0, 0))
    bspec = pl.BlockSpec((1, D), lambda i: (0, 0))
    return pl.pallas_call(
        _pool_body,
        grid=(GRID,),
        in_specs=[
            pl.BlockSpec((ROW_BLK, D), lambda i: (i, 0)),
            pl.BlockSpec((ROW_BLK, D), lambda i: (i, 0)),
            pl.BlockSpec((1, 1, ROW_BLK), lambda i: (i, 0, 0)),
            wspec, bspec, wspec, bspec, wspec, bspec,
        ],
        out_specs=pl.BlockSpec((N_GRAPHS, D), lambda i: (0, 0)),
        out_shape=jax.ShapeDtypeStruct((N_GRAPHS, D), jnp.float32),
        scratch_shapes=[pltpu.VMEM((N_GRAPHS, D), jnp.float32)],
    )(h1, agg, seg3d, w3t, b3, w4t, b4, wft, bf)


@jax.jit
def kernel(x, edge_index, batch, W1, b1, W2, b2, W3, b3, W4, b4, Wf, bf):
    src = edge_index[0].astype(jnp.int32)
    dst = edge_index[1].astype(jnp.int32)
    pad = E_PAD - N_EDGES
    src_p = jnp.concatenate([src, jnp.zeros((pad,), jnp.int32)])
    pad_dst = N_NODES + jnp.arange(pad, dtype=jnp.int32) % (AGG_ROWS - N_NODES)
    dst_p = jnp.concatenate([dst, pad_dst])
    src2d = src_p.reshape(NCHUNKS, CHUNK)
    dst2d = dst_p.reshape(NCHUNKS, CHUNK)
    zrows = jnp.zeros((ROWS_PER_SUB, D), jnp.float32)
    seg3d = batch.astype(jnp.int32).reshape(GRID, 1, ROW_BLK)

    agg1 = _sc_agg(x, src2d, dst2d, zrows)
    r1 = _mlp(x, agg1, W1.T, b1.reshape(1, D), W2.T, b2.reshape(1, D))
    agg2 = _sc_agg(r1, src2d, dst2d, zrows)
    out = _pool(r1, agg2, seg3d, W3.T, b3.reshape(1, D), W4.T, b4.reshape(1, D),
                Wf.T, bf.reshape(1, D))
    return out


# speedup vs baseline: 3.6444x; 3.6444x over previous
import functools

import jax
import jax.numpy as jnp
from jax import lax
from jax.experimental import pallas as pl
from jax.experimental.pallas import tpu as pltpu
from jax.experimental.pallas import tpu_sc as plsc

N_NODES = 10000
N_EDGES = 320000
D = 128
N_GRAPHS = 64

CHUNK = 128
N_CORES = 2
N_SUB = 16
TILES = N_CORES * N_SUB
PER_TILE = 80
NCHUNKS = TILES * PER_TILE
E_PAD = NCHUNKS * CHUNK
ROWS_PER_SUB = 632
AGG_ROWS = N_SUB * ROWS_PER_SUB

ROW_BLK = 1000
GRID = N_NODES // ROW_BLK


IBLK = 8
PT = 80
N_BLOCKS = PT // IBLK


def _sc_agg_body(x_hbm, src_hbm, dst_hbm, z_hbm, out_hbm,
                 src_v, dst_v, rows_a, rows_b, agg_sh,
                 sem_a, sem_b, sem_i, sem_j):
    c = lax.axis_index("c")
    s = lax.axis_index("s")
    tile_base = (c * N_SUB + s) * PT

    if True:
        pltpu.sync_copy(z_hbm,
                        agg_sh.at[pl.ds(s * ROWS_PER_SUB, ROWS_PER_SUB)])

        def _pref(blk, half, sem_s, sem_d):
            off = tile_base + blk * IBLK
            pltpu.async_copy(src_hbm.at[pl.ds(off, IBLK)],
                             src_v.at[pl.ds(half, IBLK)], sem_s)
            pltpu.async_copy(dst_hbm.at[pl.ds(off, IBLK)],
                             dst_v.at[pl.ds(half, IBLK)], sem_d)

        def _wait_idx(half, sem, buf):
            pltpu.make_async_copy(src_hbm.at[pl.ds(tile_base, IBLK)],
                                  buf.at[pl.ds(half, IBLK)], sem).wait()

        _pref(0, 0, sem_i, sem_j)
        _wait_idx(0, sem_i, src_v)
        pltpu.async_copy(x_hbm.at[src_v.at[0]], rows_a, sem_a)

        def _halfblock(rb, nxt_row):
            for t in range(IBLK // 2):
                r0 = rb + 2 * t
                pltpu.make_async_copy(x_hbm.at[src_v.at[r0]], rows_a,
                                      sem_a).wait()
                pltpu.async_copy(x_hbm.at[src_v.at[r0 + 1]], rows_b, sem_b)
                pltpu.sync_copy(rows_a, agg_sh.at[dst_v.at[r0]], add=True)
                pltpu.make_async_copy(x_hbm.at[src_v.at[r0 + 1]], rows_b,
                                      sem_b).wait()
                if t == IBLK // 2 - 1:
                    _wait_idx(nxt_row, sem_i, src_v)
                    nr = nxt_row
                else:
                    nr = r0 + 2
                pltpu.async_copy(x_hbm.at[src_v.at[nr]], rows_a, sem_a)
                pltpu.sync_copy(rows_b, agg_sh.at[dst_v.at[r0 + 1]],
                                add=True)

        def body(q, carry):
            _wait_idx(0, sem_j, dst_v)
            _pref(2 * q + 1, IBLK, sem_i, sem_j)
            _halfblock(0, IBLK)
            _wait_idx(IBLK, sem_j, dst_v)
            _pref(lax.rem(2 * q + 2, N_BLOCKS), 0, sem_i, sem_j)
            _halfblock(IBLK, 0)
            return carry

        lax.fori_loop(0, PT // (2 * IBLK), body, 0)
        pltpu.make_async_copy(x_hbm.at[src_v.at[0]], rows_a, sem_a).wait()
        _wait_idx(0, sem_j, dst_v)
        plsc.subcore_barrier()

        pltpu.sync_copy(agg_sh.at[pl.ds(s * ROWS_PER_SUB, ROWS_PER_SUB)],
                        out_hbm.at[c, pl.ds(s * ROWS_PER_SUB, ROWS_PER_SUB)])


def _sc_agg(x, src2d, dst2d, zrows):
    f = functools.partial(
        pl.kernel,
        out_type=jax.ShapeDtypeStruct((N_CORES, AGG_ROWS, D), jnp.float32),
        mesh=plsc.VectorSubcoreMesh(core_axis_name="c", subcore_axis_name="s"),
        scratch_types=[
            pltpu.VMEM((2 * IBLK, CHUNK), jnp.int32),
            pltpu.VMEM((2 * IBLK, CHUNK), jnp.int32),
            pltpu.VMEM((CHUNK, D), jnp.float32),
            pltpu.VMEM((CHUNK, D), jnp.float32),
            pltpu.VMEM_SHARED((AGG_ROWS, D), jnp.float32),
            pltpu.SemaphoreType.DMA,
            pltpu.SemaphoreType.DMA,
            pltpu.SemaphoreType.DMA,
            pltpu.SemaphoreType.DMA,
        ],
    )(_sc_agg_body)
    return f(x, src2d, dst2d, zrows)


def _mlp_body(x_ref, a_ref, w1_ref, b1_ref, w2_ref, b2_ref, o_ref):
    h = x_ref[...] + a_ref[0] + a_ref[1]
    h = jnp.dot(h, w1_ref[...], preferred_element_type=jnp.float32) + b1_ref[...]
    h = jnp.maximum(h, 0.0)
    h = jnp.dot(h, w2_ref[...], preferred_element_type=jnp.float32) + b2_ref[...]
    o_ref[...] = jnp.maximum(h, 0.0)


def _mlp(x, agg, w1t, b1, w2t, b2):
    wspec = pl.BlockSpec((D, D), lambda i: (0, 0))
    bspec = pl.BlockSpec((1, D), lambda i: (0, 0))
    return pl.pallas_call(
        _mlp_body,
        grid=(GRID,),
        in_specs=[
            pl.BlockSpec((ROW_BLK, D), lambda i: (i, 0)),
            pl.BlockSpec((N_CORES, ROW_BLK, D), lambda i: (0, i, 0)),
            wspec, bspec, wspec, bspec,
        ],
        out_specs=pl.BlockSpec((ROW_BLK, D), lambda i: (i, 0)),
        out_shape=jax.ShapeDtypeStruct((N_NODES, D), jnp.float32),
    )(x, agg, w1t, b1, w2t, b2)


def _pool_body(x_ref, a_ref, seg_ref, w3_ref, b3_ref, w4_ref, b4_ref,
               wf_ref, bf_ref, o_ref, acc_ref):
    i = pl.program_id(0)
    h = x_ref[...] + a_ref[0] + a_ref[1]
    h = jnp.dot(h, w3_ref[...], preferred_element_type=jnp.float32) + b3_ref[...]
    h = jnp.maximum(h, 0.0)
    h = jnp.dot(h, w4_ref[...], preferred_element_type=jnp.float32) + b4_ref[...]
    h = jnp.maximum(h, 0.0)
    seg = seg_ref[0, 0, :]
    onehot = (seg[:, None] == lax.broadcasted_iota(jnp.int32, (1, N_GRAPHS), 1))
    onehot = onehot.astype(jnp.float32)
    p = lax.dot_general(onehot, h, (((0,), (0,)), ((), ())),
                        preferred_element_type=jnp.float32)

    @pl.when(i == 0)
    def _():
        acc_ref[...] = p

    @pl.when(i > 0)
    def _():
        acc_ref[...] = acc_ref[...] + p

    @pl.when(i == GRID - 1)
    def _():
        out = jnp.dot(acc_ref[...], wf_ref[...],
                      preferred_element_type=jnp.float32) + bf_ref[...]
        o_ref[...] = jnp.maximum(out, 0.0)


def _pool(h1, agg, seg3d, w3t, b3, w4t, b4, wft, bf):
    wspec = pl.BlockSpec((D, D), lambda i: (0, 0))
    bspec = pl.BlockSpec((1, D), lambda i: (0, 0))
    return pl.pallas_call(
        _pool_body,
        grid=(GRID,),
        in_specs=[
            pl.BlockSpec((ROW_BLK, D), lambda i: (i, 0)),
            pl.BlockSpec((N_CORES, ROW_BLK, D), lambda i: (0, i, 0)),
            pl.BlockSpec((1, 1, ROW_BLK), lambda i: (i, 0, 0)),
            wspec, bspec, wspec, bspec, wspec, bspec,
        ],
        out_specs=pl.BlockSpec((N_GRAPHS, D), lambda i: (0, 0)),
        out_shape=jax.ShapeDtypeStruct((N_GRAPHS, D), jnp.float32),
        scratch_shapes=[pltpu.VMEM((N_GRAPHS, D), jnp.float32)],
    )(h1, agg, seg3d, w3t, b3, w4t, b4, wft, bf)


@jax.jit
def kernel(x, edge_index, batch, W1, b1, W2, b2, W3, b3, W4, b4, Wf, bf):
    src = edge_index[0].astype(jnp.int32)
    dst = edge_index[1].astype(jnp.int32)
    pad = E_PAD - N_EDGES
    pad_src = jnp.arange(pad, dtype=jnp.int32) % N_NODES
    src_p = jnp.concatenate([src, pad_src])
    pad_dst = N_NODES + jnp.arange(pad, dtype=jnp.int32) % (AGG_ROWS - N_NODES)
    dst_p = jnp.concatenate([dst, pad_dst])
    src2d = src_p.reshape(NCHUNKS, CHUNK)
    dst2d = dst_p.reshape(NCHUNKS, CHUNK)
    zrows = jnp.zeros((ROWS_PER_SUB, D), jnp.float32)
    seg3d = batch.astype(jnp.int32).reshape(GRID, 1, ROW_BLK)

    agg1 = _sc_agg(x, src2d, dst2d, zrows)
    r1 = _mlp(x, agg1, W1.T, b1.reshape(1, D), W2.T, b2.reshape(1, D))
    agg2 = _sc_agg(r1, src2d, dst2d, zrows)
    out = _pool(r1, agg2, seg3d, W3.T, b3.reshape(1, D), W4.T, b4.reshape(1, D),
                Wf.T, bf.reshape(1, D))
    return out
